# baseline (speedup 1.0000x reference)
import numpy as np
import jax
import jax.numpy as jnp
from jax import lax
from jax.experimental import pallas as pl
from jax.experimental.pallas import tpu as pltpu
from jax.experimental.pallas import tpu_sc as plsc

NUM_CLASSES = 21
ROIS_PER_IMAGE = 128
FG_ROIS = 32
FG_THRESH = 0.5
BG_HI = 0.5
BG_LO = 0.0
STDS = (0.1, 0.1, 0.2, 0.2)

B = 4
N = 5000
K = 20
NTOT = N + K
NPAD = 5120
LROW = NPAD // 128
NROW = B * LROW
NSTEP = NPAD // 16
KPAD = 32


def _rotl32(x, r):
    return ((x << np.uint32(r)) | (x >> np.uint32(32 - r))).astype(np.uint32)


def _threefry2x32(k0, k1, x0, x1):
    rot = [[13, 15, 26, 6], [17, 29, 16, 24]]
    ks = [np.uint32(k0), np.uint32(k1),
          np.uint32(k0) ^ np.uint32(k1) ^ np.uint32(0x1BD11BDA)]
    x0 = (x0 + ks[0]).astype(np.uint32)
    x1 = (x1 + ks[1]).astype(np.uint32)
    for i in range(5):
        for r in rot[i % 2]:
            x0 = (x0 + x1).astype(np.uint32)
            x1 = _rotl32(x1, r) ^ x0
        x0 = (x0 + ks[(i + 1) % 3]).astype(np.uint32)
        x1 = (x1 + ks[(i + 2) % 3] + np.uint32(i + 1)).astype(np.uint32)
    return x0, x1


def _const_perms():
    rows = []
    for i in range(B):
        fk0, fk1 = _threefry2x32(0, 42, np.uint32(0), np.uint32(i))
        counts = np.arange(NTOT, dtype=np.uint64)
        hi = (counts >> np.uint64(32)).astype(np.uint32)
        lo = (counts & np.uint64(0xFFFFFFFF)).astype(np.uint32)
        b0, b1 = _threefry2x32(int(fk0), int(fk1), hi, lo)
        bits = b0 ^ b1
        noise = ((bits >> np.uint32(9)) | np.uint32(0x3F800000)).view(np.float32) - np.float32(1.0)
        p = np.argsort(-noise, kind="stable").astype(np.int32)
        rows.append(np.concatenate([p, np.arange(NTOT, NPAD, dtype=np.int32)]))
    return np.stack(rows)


_PERMS = _const_perms()


def _iou_body(rx1, ry1, rx2, ry2, gx1, gy1, gx2, gy2, pmo, amo):
    ax1 = rx1[...]
    ay1 = ry1[...]
    ax2 = rx2[...]
    ay2 = ry2[...]
    aarea = (ax2 - ax1 + 1.0) * (ay2 - ay1 + 1.0)
    best = jnp.full((NROW, 128), -1.0, jnp.float32)
    bk = jnp.zeros((NROW, 128), jnp.int32)
    for k in range(1, K + 1):
        g1 = gx1[:, k:k + 1]
        h1 = gy1[:, k:k + 1]
        g2 = gx2[:, k:k + 1]
        h2 = gy2[:, k:k + 1]
        gareak = (g2 - g1 + 1.0) * (h2 - h1 + 1.0)
        iw = jnp.minimum(ax2, g2) - jnp.maximum(ax1, g1) + 1.0
        ih = jnp.minimum(ay2, h2) - jnp.maximum(ay1, h1) + 1.0
        iw = jnp.maximum(iw, 0.0)
        ih = jnp.maximum(ih, 0.0)
        inter = iw * ih
        ua = aarea + gareak - inter
        ov = inter / ua
        gtm = ov > best
        best = jnp.where(gtm, ov, best)
        bk = jnp.where(gtm, jnp.int32(k), bk)
    row = lax.broadcasted_iota(jnp.int32, (NROW, 128), 0)
    lane = lax.broadcasted_iota(jnp.int32, (NROW, 128), 1)
    eidx = lax.rem(row, LROW) * 128 + lane
    pmo[...] = jnp.where(eidx >= NTOT, -1.0, best)
    amo[...] = bk


def _sel_body(pm, am, boxes, perm, gts,
              selo,
              pm_all, am_all, px1, py1, px2, py2, perm_loc,
              lgx1, lgy1, lgx2, lgy2, lglab,
              fgsel, bgsel,
              ox1, oy1, ox2, oy2, olab, ogx1, ogy1, ogx2, ogy2, sem):
    s = lax.axis_index("s")

    @pl.when(s < B)
    def _scan():
        img = s
        cps = [pltpu.async_copy(boxes.at[img * 4 + 0], px1, sem),
               pltpu.async_copy(boxes.at[img * 4 + 1], py1, sem),
               pltpu.async_copy(boxes.at[img * 4 + 2], px2, sem),
               pltpu.async_copy(boxes.at[img * 4 + 3], py2, sem),
               pltpu.async_copy(am.at[img], am_all, sem),
               pltpu.async_copy(gts.at[img * 5 + 0], lgx1, sem),
               pltpu.async_copy(gts.at[img * 5 + 1], lgy1, sem),
               pltpu.async_copy(gts.at[img * 5 + 2], lgx2, sem),
               pltpu.async_copy(gts.at[img * 5 + 3], lgy2, sem),
               pltpu.async_copy(gts.at[img * 5 + 4], lglab, sem)]
        pltpu.sync_copy(perm.at[img], perm_loc)
        pltpu.sync_copy(pm.at[img], pm_all)
        bgsel[pl.ds(0, 16)] = jnp.zeros((16,), jnp.int32)

        zeros16 = jnp.zeros((16,), jnp.int32)
        iota16 = lax.iota(jnp.int32, 16)

        FAST_STEPS = 304
        BLK = 16

        def fast_step(t, carry):
            fg_off, bg_off = carry
            jv = perm_loc[pl.ds(t * 16, 16)]
            pmv = plsc.load_gather(pm_all, [jv])
            m_fg = pmv >= FG_THRESH
            cs_fg = plsc.cumsum(m_fg.astype(jnp.int32))
            pos_fg = fg_off + cs_fg - 1
            plsc.store_scatter(fgsel, [jnp.minimum(pos_fg, FG_ROIS - 1)], jv,
                               mask=m_fg & (pos_fg < FG_ROIS))
            pos_bg = bg_off + (iota16 - cs_fg)
            plsc.store_scatter(bgsel, [jnp.minimum(pos_bg, ROIS_PER_IMAGE - 1)], jv,
                               mask=(~m_fg) & (pos_bg < ROIS_PER_IMAGE))
            nfg = plsc.all_reduce_population_count(m_fg)
            return fg_off + nfg, bg_off + (16 - nfg)

        def blk_cond(carry):
            b, fg_off, bg_off, fg_sc, bg_sc = carry
            return (b < FAST_STEPS // BLK) & ((fg_sc < FG_ROIS) |
                                              (bg_sc < ROIS_PER_IMAGE))

        def blk_body(carry):
            b, fg_off, bg_off, _, _ = carry
            fg_off, bg_off = lax.fori_loop(b * BLK, b * BLK + BLK, fast_step,
                                           (fg_off, bg_off))
            return (b + 1, fg_off, bg_off, jnp.max(fg_off), jnp.max(bg_off))

        _, fg_off, bg_off, fg_sc, bg_sc = lax.while_loop(
            blk_cond, blk_body, (jnp.int32(0), zeros16, zeros16,
                                 jnp.int32(0), jnp.int32(0)))

        def tail_step(t, carry):
            fg_off, bg_off = carry
            jv = perm_loc[pl.ds(t * 16, 16)]
            pmv = plsc.load_gather(pm_all, [jv])
            m_fg = pmv >= FG_THRESH
            m_bg = (pmv < BG_HI) & (pmv >= BG_LO)
            pos_fg = fg_off + plsc.cumsum(m_fg.astype(jnp.int32)) - 1
            plsc.store_scatter(fgsel, [jnp.minimum(pos_fg, FG_ROIS - 1)], jv,
                               mask=m_fg & (pos_fg < FG_ROIS))
            pos_bg = bg_off + plsc.cumsum(m_bg.astype(jnp.int32)) - 1
            plsc.store_scatter(bgsel, [jnp.minimum(pos_bg, ROIS_PER_IMAGE - 1)], jv,
                               mask=m_bg & (pos_bg < ROIS_PER_IMAGE))
            fg_off = fg_off + plsc.all_reduce_population_count(m_fg)
            bg_off = bg_off + plsc.all_reduce_population_count(m_bg)
            return fg_off, bg_off

        fg_off, bg_off = lax.cond(
            (fg_sc < FG_ROIS) | (bg_sc < ROIS_PER_IMAGE),
            lambda: lax.fori_loop(FAST_STEPS, NSTEP, tail_step,
                                  (fg_off, bg_off)),
            lambda: (fg_off, bg_off))

        fg_this = jnp.minimum(fg_off, FG_ROIS)
        bg_mod = jnp.minimum(jnp.maximum(bg_off, 1), ROIS_PER_IMAGE)

        for cp in cps:
            cp.wait()

        for t in range(ROIS_PER_IMAGE // 16):
            iv = t * 16 + lax.iota(jnp.int32, 16)
            m_isfg = iv < fg_this
            fsel = plsc.load_gather(fgsel, [jnp.minimum(iv, FG_ROIS - 1)])
            bslot = lax.rem(jnp.maximum(iv - fg_this, 0), bg_mod)
            bsel = plsc.load_gather(bgsel, [bslot])
            keep = jnp.where(m_isfg, fsel, bsel)
            amk = plsc.load_gather(am_all, [keep])
            labv = plsc.load_gather(lglab, [amk])
            sl = pl.ds(t * 16, 16)
            ox1[sl] = plsc.load_gather(px1, [keep])
            oy1[sl] = plsc.load_gather(py1, [keep])
            ox2[sl] = plsc.load_gather(px2, [keep])
            oy2[sl] = plsc.load_gather(py2, [keep])
            olab[sl] = jnp.where(m_isfg, labv, 0.0)
            ogx1[sl] = plsc.load_gather(lgx1, [amk])
            ogy1[sl] = plsc.load_gather(lgy1, [amk])
            ogx2[sl] = plsc.load_gather(lgx2, [amk])
            ogy2[sl] = plsc.load_gather(lgy2, [amk])

        pltpu.sync_copy(ox1, selo.at[0 * B + img])
        pltpu.sync_copy(oy1, selo.at[1 * B + img])
        pltpu.sync_copy(ox2, selo.at[2 * B + img])
        pltpu.sync_copy(oy2, selo.at[3 * B + img])
        pltpu.sync_copy(olab, selo.at[4 * B + img])
        pltpu.sync_copy(ogx1, selo.at[5 * B + img])
        pltpu.sync_copy(ogy1, selo.at[6 * B + img])
        pltpu.sync_copy(ogx2, selo.at[7 * B + img])
        pltpu.sync_copy(ogy2, selo.at[8 * B + img])


def _make_sel_call():
    f32 = jnp.float32
    i32 = jnp.int32
    out = [jax.ShapeDtypeStruct((9 * B, ROIS_PER_IMAGE), f32)]
    scratch = [
        pltpu.VMEM((NPAD,), f32),
        pltpu.VMEM((NPAD,), i32),
        pltpu.VMEM((NPAD,), f32),
        pltpu.VMEM((NPAD,), f32),
        pltpu.VMEM((NPAD,), f32),
        pltpu.VMEM((NPAD,), f32),
        pltpu.VMEM((NPAD,), i32),
        pltpu.VMEM((KPAD,), f32),
        pltpu.VMEM((KPAD,), f32),
        pltpu.VMEM((KPAD,), f32),
        pltpu.VMEM((KPAD,), f32),
        pltpu.VMEM((KPAD,), f32),
        pltpu.VMEM((FG_ROIS,), i32),
        pltpu.VMEM((ROIS_PER_IMAGE,), i32),
        pltpu.VMEM((ROIS_PER_IMAGE,), f32),
        pltpu.VMEM((ROIS_PER_IMAGE,), f32),
        pltpu.VMEM((ROIS_PER_IMAGE,), f32),
        pltpu.VMEM((ROIS_PER_IMAGE,), f32),
        pltpu.VMEM((ROIS_PER_IMAGE,), f32),
        pltpu.VMEM((ROIS_PER_IMAGE,), f32),
        pltpu.VMEM((ROIS_PER_IMAGE,), f32),
        pltpu.VMEM((ROIS_PER_IMAGE,), f32),
        pltpu.VMEM((ROIS_PER_IMAGE,), f32),
        pltpu.SemaphoreType.DMA,
    ]
    mesh = plsc.VectorSubcoreMesh(core_axis_name="c", subcore_axis_name="s",
                                  num_cores=1, num_subcores=16)
    return pl.kernel(_sel_body, out_type=out, mesh=mesh, scratch_types=scratch,
                     compiler_params=pltpu.CompilerParams(needs_layout_passes=False))


def _tc_body(selp, rois, labels, tgts, ins, outs):
    x1 = selp[0 * B:1 * B]
    y1 = selp[1 * B:2 * B]
    x2 = selp[2 * B:3 * B]
    y2 = selp[3 * B:4 * B]
    ew = x2 - x1 + 1.0
    eh = y2 - y1 + 1.0
    ecx = x1 + 0.5 * ew
    ecy = y1 + 0.5 * eh
    g1 = selp[5 * B:6 * B]
    h1 = selp[6 * B:7 * B]
    g2 = selp[7 * B:8 * B]
    h2 = selp[8 * B:9 * B]
    gw = g2 - g1 + 1.0
    gh = h2 - h1 + 1.0
    gcx = g1 + 0.5 * gw
    gcy = h1 + 0.5 * gh
    dx = ((gcx - ecx) / ew) / STDS[0]
    dy = ((gcy - ecy) / eh) / STDS[1]
    dw = jnp.log(gw / ew) / STDS[2]
    dh = jnp.log(gh / eh) / STDS[3]
    lb = selp[4 * B:5 * B]
    fg = lb > 0.0
    w = jnp.where(fg, 1.0, 0.0)
    col0 = lax.broadcasted_iota(jnp.int32, (B, ROIS_PER_IMAGE), 0).astype(jnp.float32)
    rois[...] = jnp.stack([col0, x1, y1, x2, y2], axis=-1)
    labels[...] = lb
    tgts[...] = jnp.stack([jnp.where(fg, dx, 0.0), jnp.where(fg, dy, 0.0),
                           jnp.where(fg, dw, 0.0), jnp.where(fg, dh, 0.0)],
                          axis=-1)
    wh = jnp.stack([w, w, w, w], axis=-1)
    ins[...] = wh
    outs[...] = wh


def kernel(all_rois, gt_boxes, num_boxes):
    f32 = jnp.float32
    pad = ((0, 0), (0, NPAD - NTOT))
    rx1 = jnp.pad(jnp.concatenate([all_rois[:, :, 1], gt_boxes[:, :, 0]], axis=1), pad)
    ry1 = jnp.pad(jnp.concatenate([all_rois[:, :, 2], gt_boxes[:, :, 1]], axis=1), pad)
    rx2 = jnp.pad(jnp.concatenate([all_rois[:, :, 3], gt_boxes[:, :, 2]], axis=1), pad)
    ry2 = jnp.pad(jnp.concatenate([all_rois[:, :, 4], gt_boxes[:, :, 3]], axis=1), pad)
    kp = ((0, 0), (1, KPAD - K - 1))
    gx1 = jnp.pad(gt_boxes[:, :, 0], kp)
    gy1 = jnp.pad(gt_boxes[:, :, 1], kp)
    gx2 = jnp.pad(gt_boxes[:, :, 2], kp)
    gy2 = jnp.pad(gt_boxes[:, :, 3], kp)
    glab = jnp.pad(gt_boxes[:, :, 4], kp)
    perm = jnp.asarray(_PERMS)

    rq = [a.reshape(NROW, 128) for a in (rx1, ry1, rx2, ry2)]
    ge = [jnp.repeat(a, LROW, axis=0) for a in (gx1, gy1, gx2, gy2)]
    pmq, amq = pl.pallas_call(
        _iou_body,
        out_shape=[jax.ShapeDtypeStruct((NROW, 128), f32),
                   jax.ShapeDtypeStruct((NROW, 128), jnp.int32)])(*rq, *ge)
    pm = pmq.reshape(B, NPAD)
    am = amq.reshape(B, NPAD)

    boxes = jnp.stack([rx1, ry1, rx2, ry2], axis=1).reshape(B * 4, NPAD)
    gts = jnp.stack([gx1, gy1, gx2, gy2, glab], axis=1).reshape(B * 5, KPAD)

    sel = _make_sel_call()
    (selp,) = sel(pm, am, boxes, perm, gts)

    rois, labels, bbox_targets, bbox_inside, bbox_outside = pl.pallas_call(
        _tc_body,
        out_shape=[jax.ShapeDtypeStruct((B, ROIS_PER_IMAGE, 5), f32),
                   jax.ShapeDtypeStruct((B, ROIS_PER_IMAGE), f32),
                   jax.ShapeDtypeStruct((B, ROIS_PER_IMAGE, 4), f32),
                   jax.ShapeDtypeStruct((B, ROIS_PER_IMAGE, 4), f32),
                   jax.ShapeDtypeStruct((B, ROIS_PER_IMAGE, 4), f32)],
    )(selp)
    return rois, labels, bbox_targets, bbox_inside, bbox_outside


# speedup vs baseline: 1.0325x; 1.0325x over previous
import numpy as np
import jax
import jax.numpy as jnp
from jax import lax
from jax.experimental import pallas as pl
from jax.experimental.pallas import tpu as pltpu
from jax.experimental.pallas import tpu_sc as plsc

NUM_CLASSES = 21
ROIS_PER_IMAGE = 128
FG_ROIS = 32
FG_THRESH = 0.5
BG_HI = 0.5
BG_LO = 0.0
STDS = (0.1, 0.1, 0.2, 0.2)

B = 4
N = 5000
K = 20
NTOT = N + K
NPAD = 5120
LROW = NPAD // 128
NROW = B * LROW
NSTEP = NPAD // 16
KPAD = 32


def _rotl32(x, r):
    return ((x << np.uint32(r)) | (x >> np.uint32(32 - r))).astype(np.uint32)


def _threefry2x32(k0, k1, x0, x1):
    rot = [[13, 15, 26, 6], [17, 29, 16, 24]]
    ks = [np.uint32(k0), np.uint32(k1),
          np.uint32(k0) ^ np.uint32(k1) ^ np.uint32(0x1BD11BDA)]
    x0 = (x0 + ks[0]).astype(np.uint32)
    x1 = (x1 + ks[1]).astype(np.uint32)
    for i in range(5):
        for r in rot[i % 2]:
            x0 = (x0 + x1).astype(np.uint32)
            x1 = _rotl32(x1, r) ^ x0
        x0 = (x0 + ks[(i + 1) % 3]).astype(np.uint32)
        x1 = (x1 + ks[(i + 2) % 3] + np.uint32(i + 1)).astype(np.uint32)
    return x0, x1


def _const_perms():
    rows = []
    for i in range(B):
        fk0, fk1 = _threefry2x32(0, 42, np.uint32(0), np.uint32(i))
        counts = np.arange(NTOT, dtype=np.uint64)
        hi = (counts >> np.uint64(32)).astype(np.uint32)
        lo = (counts & np.uint64(0xFFFFFFFF)).astype(np.uint32)
        b0, b1 = _threefry2x32(int(fk0), int(fk1), hi, lo)
        bits = b0 ^ b1
        noise = ((bits >> np.uint32(9)) | np.uint32(0x3F800000)).view(np.float32) - np.float32(1.0)
        p = np.argsort(-noise, kind="stable").astype(np.int32)
        rows.append(np.concatenate([p, np.arange(NTOT, NPAD, dtype=np.int32)]))
    return np.stack(rows)


_PERMS = _const_perms()


def _iou_body(rx1, ry1, rx2, ry2, gx1, gy1, gx2, gy2, pmo, amo):
    ax1 = rx1[...]
    ay1 = ry1[...]
    ax2 = rx2[...]
    ay2 = ry2[...]
    aarea = (ax2 - ax1 + 1.0) * (ay2 - ay1 + 1.0)
    best = jnp.full((NROW, 128), -1.0, jnp.float32)
    bk = jnp.zeros((NROW, 128), jnp.int32)
    for k in range(1, K + 1):
        g1 = gx1[:, k:k + 1]
        h1 = gy1[:, k:k + 1]
        g2 = gx2[:, k:k + 1]
        h2 = gy2[:, k:k + 1]
        gareak = (g2 - g1 + 1.0) * (h2 - h1 + 1.0)
        iw = jnp.minimum(ax2, g2) - jnp.maximum(ax1, g1) + 1.0
        ih = jnp.minimum(ay2, h2) - jnp.maximum(ay1, h1) + 1.0
        iw = jnp.maximum(iw, 0.0)
        ih = jnp.maximum(ih, 0.0)
        inter = iw * ih
        ua = aarea + gareak - inter
        ov = inter / ua
        gtm = ov > best
        best = jnp.where(gtm, ov, best)
        bk = jnp.where(gtm, jnp.int32(k), bk)
    row = lax.broadcasted_iota(jnp.int32, (NROW, 128), 0)
    lane = lax.broadcasted_iota(jnp.int32, (NROW, 128), 1)
    eidx = lax.rem(row, LROW) * 128 + lane
    pmo[...] = jnp.where(eidx >= NTOT, -1.0, best)
    amo[...] = bk


def _sel_body(pm, am, rx1, ry1, rx2, ry2, perm, gx1, gy1, gx2, gy2, glab,
              selo,
              pm_all, am_all, px1, py1, px2, py2, perm_loc,
              lgx1, lgy1, lgx2, lgy2, lglab,
              fgsel, bgsel,
              ox1, oy1, ox2, oy2, olab, ogx1, ogy1, ogx2, ogy2, sem, sem2):
    s = lax.axis_index("s")

    @pl.when(s < B)
    def _scan():
        img = s
        cps = [pltpu.async_copy(rx1.at[img], px1, sem),
               pltpu.async_copy(ry1.at[img], py1, sem),
               pltpu.async_copy(rx2.at[img], px2, sem),
               pltpu.async_copy(ry2.at[img], py2, sem),
               pltpu.async_copy(am.at[img], am_all, sem),
               pltpu.async_copy(gx1.at[img], lgx1, sem),
               pltpu.async_copy(gy1.at[img], lgy1, sem),
               pltpu.async_copy(gx2.at[img], lgx2, sem),
               pltpu.async_copy(gy2.at[img], lgy2, sem),
               pltpu.async_copy(glab.at[img], lglab, sem)]
        cp_perm = pltpu.async_copy(perm.at[img], perm_loc, sem2)
        cp_pm = pltpu.async_copy(pm.at[img], pm_all, sem2)
        bgsel[pl.ds(0, 16)] = jnp.zeros((16,), jnp.int32)
        cp_perm.wait()
        cp_pm.wait()

        zeros16 = jnp.zeros((16,), jnp.int32)
        iota16 = lax.iota(jnp.int32, 16)

        FAST_STEPS = 304
        BLK = 16

        def fast_one(t, fg_off, bg_off):
            jv = perm_loc[pl.ds(t * 16, 16)]
            pmv = plsc.load_gather(pm_all, [jv])
            m_fg = pmv >= FG_THRESH
            cs_fg = plsc.cumsum(m_fg.astype(jnp.int32))
            pos_fg = fg_off + cs_fg - 1
            plsc.store_scatter(fgsel, [jnp.minimum(pos_fg, FG_ROIS - 1)], jv,
                               mask=m_fg & (pos_fg < FG_ROIS))
            pos_bg = bg_off + (iota16 - cs_fg)
            plsc.store_scatter(bgsel, [jnp.minimum(pos_bg, ROIS_PER_IMAGE - 1)], jv,
                               mask=(~m_fg) & (pos_bg < ROIS_PER_IMAGE))
            nfg = plsc.all_reduce_population_count(m_fg)
            return fg_off + nfg, bg_off + (16 - nfg)

        def fast_pair(u, carry):
            fg_off, bg_off = carry
            fg_off, bg_off = fast_one(u * 2, fg_off, bg_off)
            fg_off, bg_off = fast_one(u * 2 + 1, fg_off, bg_off)
            return fg_off, bg_off

        def blk_cond(carry):
            b, fg_off, bg_off, fg_sc, bg_sc = carry
            return (b < FAST_STEPS // BLK) & ((fg_sc < FG_ROIS) |
                                              (bg_sc < ROIS_PER_IMAGE))

        def blk_body(carry):
            b, fg_off, bg_off, _, _ = carry
            fg_off, bg_off = lax.fori_loop(b * (BLK // 2), (b + 1) * (BLK // 2),
                                           fast_pair, (fg_off, bg_off))
            return (b + 1, fg_off, bg_off, jnp.max(fg_off), jnp.max(bg_off))

        _, fg_off, bg_off, fg_sc, bg_sc = lax.while_loop(
            blk_cond, blk_body, (jnp.int32(0), zeros16, zeros16,
                                 jnp.int32(0), jnp.int32(0)))

        def tail_step(t, carry):
            fg_off, bg_off = carry
            jv = perm_loc[pl.ds(t * 16, 16)]
            pmv = plsc.load_gather(pm_all, [jv])
            m_fg = pmv >= FG_THRESH
            m_bg = (pmv < BG_HI) & (pmv >= BG_LO)
            pos_fg = fg_off + plsc.cumsum(m_fg.astype(jnp.int32)) - 1
            plsc.store_scatter(fgsel, [jnp.minimum(pos_fg, FG_ROIS - 1)], jv,
                               mask=m_fg & (pos_fg < FG_ROIS))
            pos_bg = bg_off + plsc.cumsum(m_bg.astype(jnp.int32)) - 1
            plsc.store_scatter(bgsel, [jnp.minimum(pos_bg, ROIS_PER_IMAGE - 1)], jv,
                               mask=m_bg & (pos_bg < ROIS_PER_IMAGE))
            fg_off = fg_off + plsc.all_reduce_population_count(m_fg)
            bg_off = bg_off + plsc.all_reduce_population_count(m_bg)
            return fg_off, bg_off

        fg_off, bg_off = lax.cond(
            (fg_sc < FG_ROIS) | (bg_sc < ROIS_PER_IMAGE),
            lambda: lax.fori_loop(FAST_STEPS, NSTEP, tail_step,
                                  (fg_off, bg_off)),
            lambda: (fg_off, bg_off))

        fg_this = jnp.minimum(fg_off, FG_ROIS)
        bg_mod = jnp.minimum(jnp.maximum(bg_off, 1), ROIS_PER_IMAGE)

        for cp in cps:
            cp.wait()

        for t in range(ROIS_PER_IMAGE // 16):
            iv = t * 16 + lax.iota(jnp.int32, 16)
            m_isfg = iv < fg_this
            fsel = plsc.load_gather(fgsel, [jnp.minimum(iv, FG_ROIS - 1)])
            bslot = lax.rem(jnp.maximum(iv - fg_this, 0), bg_mod)
            bsel = plsc.load_gather(bgsel, [bslot])
            keep = jnp.where(m_isfg, fsel, bsel)
            amk = plsc.load_gather(am_all, [keep])
            labv = plsc.load_gather(lglab, [amk])
            sl = pl.ds(t * 16, 16)
            ox1[sl] = plsc.load_gather(px1, [keep])
            oy1[sl] = plsc.load_gather(py1, [keep])
            ox2[sl] = plsc.load_gather(px2, [keep])
            oy2[sl] = plsc.load_gather(py2, [keep])
            olab[sl] = jnp.where(m_isfg, labv, 0.0)
            ogx1[sl] = plsc.load_gather(lgx1, [amk])
            ogy1[sl] = plsc.load_gather(lgy1, [amk])
            ogx2[sl] = plsc.load_gather(lgx2, [amk])
            ogy2[sl] = plsc.load_gather(lgy2, [amk])

        pltpu.sync_copy(ox1, selo.at[0 * B + img])
        pltpu.sync_copy(oy1, selo.at[1 * B + img])
        pltpu.sync_copy(ox2, selo.at[2 * B + img])
        pltpu.sync_copy(oy2, selo.at[3 * B + img])
        pltpu.sync_copy(olab, selo.at[4 * B + img])
        pltpu.sync_copy(ogx1, selo.at[5 * B + img])
        pltpu.sync_copy(ogy1, selo.at[6 * B + img])
        pltpu.sync_copy(ogx2, selo.at[7 * B + img])
        pltpu.sync_copy(ogy2, selo.at[8 * B + img])


def _make_sel_call():
    f32 = jnp.float32
    i32 = jnp.int32
    out = [jax.ShapeDtypeStruct((9 * B, ROIS_PER_IMAGE), f32)]
    scratch = [
        pltpu.VMEM((NPAD,), f32),
        pltpu.VMEM((NPAD,), i32),
        pltpu.VMEM((NPAD,), f32),
        pltpu.VMEM((NPAD,), f32),
        pltpu.VMEM((NPAD,), f32),
        pltpu.VMEM((NPAD,), f32),
        pltpu.VMEM((NPAD,), i32),
        pltpu.VMEM((KPAD,), f32),
        pltpu.VMEM((KPAD,), f32),
        pltpu.VMEM((KPAD,), f32),
        pltpu.VMEM((KPAD,), f32),
        pltpu.VMEM((KPAD,), f32),
        pltpu.VMEM((FG_ROIS,), i32),
        pltpu.VMEM((ROIS_PER_IMAGE,), i32),
        pltpu.VMEM((ROIS_PER_IMAGE,), f32),
        pltpu.VMEM((ROIS_PER_IMAGE,), f32),
        pltpu.VMEM((ROIS_PER_IMAGE,), f32),
        pltpu.VMEM((ROIS_PER_IMAGE,), f32),
        pltpu.VMEM((ROIS_PER_IMAGE,), f32),
        pltpu.VMEM((ROIS_PER_IMAGE,), f32),
        pltpu.VMEM((ROIS_PER_IMAGE,), f32),
        pltpu.VMEM((ROIS_PER_IMAGE,), f32),
        pltpu.VMEM((ROIS_PER_IMAGE,), f32),
        pltpu.SemaphoreType.DMA,
        pltpu.SemaphoreType.DMA,
    ]
    mesh = plsc.VectorSubcoreMesh(core_axis_name="c", subcore_axis_name="s",
                                  num_cores=1, num_subcores=16)
    return pl.kernel(_sel_body, out_type=out, mesh=mesh, scratch_types=scratch,
                     compiler_params=pltpu.CompilerParams(needs_layout_passes=False))


def _tc_body(selp, rois, labels, tgts, ins, outs):
    x1 = selp[0 * B:1 * B]
    y1 = selp[1 * B:2 * B]
    x2 = selp[2 * B:3 * B]
    y2 = selp[3 * B:4 * B]
    ew = x2 - x1 + 1.0
    eh = y2 - y1 + 1.0
    ecx = x1 + 0.5 * ew
    ecy = y1 + 0.5 * eh
    g1 = selp[5 * B:6 * B]
    h1 = selp[6 * B:7 * B]
    g2 = selp[7 * B:8 * B]
    h2 = selp[8 * B:9 * B]
    gw = g2 - g1 + 1.0
    gh = h2 - h1 + 1.0
    gcx = g1 + 0.5 * gw
    gcy = h1 + 0.5 * gh
    dx = ((gcx - ecx) / ew) / STDS[0]
    dy = ((gcy - ecy) / eh) / STDS[1]
    dw = jnp.log(gw / ew) / STDS[2]
    dh = jnp.log(gh / eh) / STDS[3]
    lb = selp[4 * B:5 * B]
    fg = lb > 0.0
    w = jnp.where(fg, 1.0, 0.0)
    col0 = lax.broadcasted_iota(jnp.int32, (B, ROIS_PER_IMAGE), 0).astype(jnp.float32)
    rois[...] = jnp.stack([col0, x1, y1, x2, y2], axis=-1)
    labels[...] = lb
    tgts[...] = jnp.stack([jnp.where(fg, dx, 0.0), jnp.where(fg, dy, 0.0),
                           jnp.where(fg, dw, 0.0), jnp.where(fg, dh, 0.0)],
                          axis=-1)
    wh = jnp.stack([w, w, w, w], axis=-1)
    ins[...] = wh
    outs[...] = wh


def kernel(all_rois, gt_boxes, num_boxes):
    f32 = jnp.float32
    pad = ((0, 0), (0, NPAD - NTOT))
    rx1 = jnp.pad(jnp.concatenate([all_rois[:, :, 1], gt_boxes[:, :, 0]], axis=1), pad)
    ry1 = jnp.pad(jnp.concatenate([all_rois[:, :, 2], gt_boxes[:, :, 1]], axis=1), pad)
    rx2 = jnp.pad(jnp.concatenate([all_rois[:, :, 3], gt_boxes[:, :, 2]], axis=1), pad)
    ry2 = jnp.pad(jnp.concatenate([all_rois[:, :, 4], gt_boxes[:, :, 3]], axis=1), pad)
    kp = ((0, 0), (1, KPAD - K - 1))
    gx1 = jnp.pad(gt_boxes[:, :, 0], kp)
    gy1 = jnp.pad(gt_boxes[:, :, 1], kp)
    gx2 = jnp.pad(gt_boxes[:, :, 2], kp)
    gy2 = jnp.pad(gt_boxes[:, :, 3], kp)
    glab = jnp.pad(gt_boxes[:, :, 4], kp)
    perm = jnp.asarray(_PERMS)

    rq = [a.reshape(NROW, 128) for a in (rx1, ry1, rx2, ry2)]
    ge = [jnp.repeat(a, LROW, axis=0) for a in (gx1, gy1, gx2, gy2)]
    pmq, amq = pl.pallas_call(
        _iou_body,
        out_shape=[jax.ShapeDtypeStruct((NROW, 128), f32),
                   jax.ShapeDtypeStruct((NROW, 128), jnp.int32)])(*rq, *ge)
    pm = pmq.reshape(B, NPAD)
    am = amq.reshape(B, NPAD)

    sel = _make_sel_call()
    (selp,) = sel(pm, am, rx1, ry1, rx2, ry2, perm, gx1, gy1, gx2, gy2, glab)

    rois, labels, bbox_targets, bbox_inside, bbox_outside = pl.pallas_call(
        _tc_body,
        out_shape=[jax.ShapeDtypeStruct((B, ROIS_PER_IMAGE, 5), f32),
                   jax.ShapeDtypeStruct((B, ROIS_PER_IMAGE), f32),
                   jax.ShapeDtypeStruct((B, ROIS_PER_IMAGE, 4), f32),
                   jax.ShapeDtypeStruct((B, ROIS_PER_IMAGE, 4), f32),
                   jax.ShapeDtypeStruct((B, ROIS_PER_IMAGE, 4), f32)],
    )(selp)
    return rois, labels, bbox_targets, bbox_inside, bbox_outside


# speedup vs baseline: 1.0450x; 1.0122x over previous
import numpy as np
import jax
import jax.numpy as jnp
from jax import lax
from jax.experimental import pallas as pl
from jax.experimental.pallas import tpu as pltpu
from jax.experimental.pallas import tpu_sc as plsc

NUM_CLASSES = 21
ROIS_PER_IMAGE = 128
FG_ROIS = 32
FG_THRESH = 0.5
BG_HI = 0.5
BG_LO = 0.0
STDS = (0.1, 0.1, 0.2, 0.2)

B = 4
N = 5000
K = 20
NTOT = N + K
NPAD = 5120
LROW = NPAD // 128
NROW = B * LROW
NSTEP = NPAD // 16
KPAD = 32


def _rotl32(x, r):
    return ((x << np.uint32(r)) | (x >> np.uint32(32 - r))).astype(np.uint32)


def _threefry2x32(k0, k1, x0, x1):
    rot = [[13, 15, 26, 6], [17, 29, 16, 24]]
    ks = [np.uint32(k0), np.uint32(k1),
          np.uint32(k0) ^ np.uint32(k1) ^ np.uint32(0x1BD11BDA)]
    x0 = (x0 + ks[0]).astype(np.uint32)
    x1 = (x1 + ks[1]).astype(np.uint32)
    for i in range(5):
        for r in rot[i % 2]:
            x0 = (x0 + x1).astype(np.uint32)
            x1 = _rotl32(x1, r) ^ x0
        x0 = (x0 + ks[(i + 1) % 3]).astype(np.uint32)
        x1 = (x1 + ks[(i + 2) % 3] + np.uint32(i + 1)).astype(np.uint32)
    return x0, x1


def _const_perms():
    rows = []
    for i in range(B):
        fk0, fk1 = _threefry2x32(0, 42, np.uint32(0), np.uint32(i))
        counts = np.arange(NTOT, dtype=np.uint64)
        hi = (counts >> np.uint64(32)).astype(np.uint32)
        lo = (counts & np.uint64(0xFFFFFFFF)).astype(np.uint32)
        b0, b1 = _threefry2x32(int(fk0), int(fk1), hi, lo)
        bits = b0 ^ b1
        noise = ((bits >> np.uint32(9)) | np.uint32(0x3F800000)).view(np.float32) - np.float32(1.0)
        p = np.argsort(-noise, kind="stable").astype(np.int32)
        rows.append(np.concatenate([p, np.arange(NTOT, NPAD, dtype=np.int32)]))
    return np.stack(rows)


_PERMS = _const_perms()


def _iou_body(rx1, ry1, rx2, ry2, gx1, gy1, gx2, gy2, pmo, amo):
    ax1 = rx1[...]
    ay1 = ry1[...]
    ax2 = rx2[...]
    ay2 = ry2[...]
    aarea = (ax2 - ax1 + 1.0) * (ay2 - ay1 + 1.0)
    best = jnp.full((NROW, 128), -1.0, jnp.float32)
    bk = jnp.zeros((NROW, 128), jnp.int32)
    for k in range(1, K + 1):
        g1 = gx1[:, k:k + 1]
        h1 = gy1[:, k:k + 1]
        g2 = gx2[:, k:k + 1]
        h2 = gy2[:, k:k + 1]
        gareak = (g2 - g1 + 1.0) * (h2 - h1 + 1.0)
        iw = jnp.minimum(ax2, g2) - jnp.maximum(ax1, g1) + 1.0
        ih = jnp.minimum(ay2, h2) - jnp.maximum(ay1, h1) + 1.0
        iw = jnp.maximum(iw, 0.0)
        ih = jnp.maximum(ih, 0.0)
        inter = iw * ih
        ua = aarea + gareak - inter
        ov = inter / ua
        gtm = ov > best
        best = jnp.where(gtm, ov, best)
        bk = jnp.where(gtm, jnp.int32(k), bk)
    row = lax.broadcasted_iota(jnp.int32, (NROW, 128), 0)
    lane = lax.broadcasted_iota(jnp.int32, (NROW, 128), 1)
    eidx = lax.rem(row, LROW) * 128 + lane
    pmo[...] = jnp.where(eidx >= NTOT, -1.0, best)
    amo[...] = bk


def _sel_body(pm, am, rx1, ry1, rx2, ry2, perm, gx1, gy1, gx2, gy2, glab,
              selo,
              pm_all, am_all, px1, py1, px2, py2, perm_loc,
              lgx1, lgy1, lgx2, lgy2, lglab,
              fgsel, bgsel,
              ox1, oy1, ox2, oy2, olab, ogx1, ogy1, ogx2, ogy2, sem, sem2):
    s = lax.axis_index("s")

    @pl.when(s < B)
    def _scan():
        img = s
        cps = [pltpu.async_copy(rx1.at[img], px1, sem),
               pltpu.async_copy(ry1.at[img], py1, sem),
               pltpu.async_copy(rx2.at[img], px2, sem),
               pltpu.async_copy(ry2.at[img], py2, sem),
               pltpu.async_copy(am.at[img], am_all, sem),
               pltpu.async_copy(gx1.at[img], lgx1, sem),
               pltpu.async_copy(gy1.at[img], lgy1, sem),
               pltpu.async_copy(gx2.at[img], lgx2, sem),
               pltpu.async_copy(gy2.at[img], lgy2, sem),
               pltpu.async_copy(glab.at[img], lglab, sem)]
        cp_perm = pltpu.async_copy(perm.at[img], perm_loc, sem2)
        cp_pm = pltpu.async_copy(pm.at[img], pm_all, sem2)
        bgsel[pl.ds(0, 16)] = jnp.zeros((16,), jnp.int32)
        cp_perm.wait()
        cp_pm.wait()

        zeros16 = jnp.zeros((16,), jnp.int32)
        iota16 = lax.iota(jnp.int32, 16)

        FAST_STEPS = 304
        BLK = 16

        def fast_one(t, fg_off, bg_off):
            jv = perm_loc[pl.ds(t * 16, 16)]
            pmv = plsc.load_gather(pm_all, [jv])
            m_fg = pmv >= FG_THRESH
            cs_fg = plsc.cumsum(m_fg.astype(jnp.int32))
            pos_fg = fg_off + cs_fg - 1
            plsc.store_scatter(fgsel, [jnp.minimum(pos_fg, FG_ROIS - 1)], jv,
                               mask=m_fg & (pos_fg < FG_ROIS))
            pos_bg = bg_off + (iota16 - cs_fg)
            plsc.store_scatter(bgsel, [jnp.minimum(pos_bg, ROIS_PER_IMAGE - 1)], jv,
                               mask=(~m_fg) & (pos_bg < ROIS_PER_IMAGE))
            nfg = plsc.all_reduce_population_count(m_fg)
            return fg_off + nfg, bg_off + (16 - nfg)

        def fast_pair(u, carry):
            fg_off, bg_off = carry
            fg_off, bg_off = fast_one(u * 2, fg_off, bg_off)
            fg_off, bg_off = fast_one(u * 2 + 1, fg_off, bg_off)
            return fg_off, bg_off

        def blk_cond(carry):
            b, fg_off, bg_off, fg_sc, bg_sc = carry
            return (b < FAST_STEPS // BLK) & ((fg_sc < FG_ROIS) |
                                              (bg_sc < ROIS_PER_IMAGE))

        def blk_body(carry):
            b, fg_off, bg_off, _, _ = carry
            fg_off, bg_off = lax.fori_loop(b * (BLK // 2), (b + 1) * (BLK // 2),
                                           fast_pair, (fg_off, bg_off))
            return (b + 1, fg_off, bg_off, jnp.max(fg_off), jnp.max(bg_off))

        _, fg_off, bg_off, fg_sc, bg_sc = lax.while_loop(
            blk_cond, blk_body, (jnp.int32(0), zeros16, zeros16,
                                 jnp.int32(0), jnp.int32(0)))

        def tail_step(t, carry):
            fg_off, bg_off = carry
            jv = perm_loc[pl.ds(t * 16, 16)]
            pmv = plsc.load_gather(pm_all, [jv])
            m_fg = pmv >= FG_THRESH
            m_bg = (pmv < BG_HI) & (pmv >= BG_LO)
            pos_fg = fg_off + plsc.cumsum(m_fg.astype(jnp.int32)) - 1
            plsc.store_scatter(fgsel, [jnp.minimum(pos_fg, FG_ROIS - 1)], jv,
                               mask=m_fg & (pos_fg < FG_ROIS))
            pos_bg = bg_off + plsc.cumsum(m_bg.astype(jnp.int32)) - 1
            plsc.store_scatter(bgsel, [jnp.minimum(pos_bg, ROIS_PER_IMAGE - 1)], jv,
                               mask=m_bg & (pos_bg < ROIS_PER_IMAGE))
            fg_off = fg_off + plsc.all_reduce_population_count(m_fg)
            bg_off = bg_off + plsc.all_reduce_population_count(m_bg)
            return fg_off, bg_off

        fg_off, bg_off = lax.cond(
            (fg_sc < FG_ROIS) | (bg_sc < ROIS_PER_IMAGE),
            lambda: lax.fori_loop(FAST_STEPS, NSTEP, tail_step,
                                  (fg_off, bg_off)),
            lambda: (fg_off, bg_off))

        fg_this = jnp.minimum(fg_off, FG_ROIS)
        bg_mod = jnp.minimum(jnp.maximum(bg_off, 1), ROIS_PER_IMAGE)

        for cp in cps:
            cp.wait()

        for t in range(ROIS_PER_IMAGE // 16):
            iv = t * 16 + lax.iota(jnp.int32, 16)
            m_isfg = iv < fg_this
            fsel = plsc.load_gather(fgsel, [jnp.minimum(iv, FG_ROIS - 1)])
            bslot = lax.rem(jnp.maximum(iv - fg_this, 0), bg_mod)
            bsel = plsc.load_gather(bgsel, [bslot])
            keep = jnp.where(m_isfg, fsel, bsel)
            amk = plsc.load_gather(am_all, [keep])
            labv = plsc.load_gather(lglab, [amk])
            sl = pl.ds(t * 16, 16)
            ox1[sl] = plsc.load_gather(px1, [keep])
            oy1[sl] = plsc.load_gather(py1, [keep])
            ox2[sl] = plsc.load_gather(px2, [keep])
            oy2[sl] = plsc.load_gather(py2, [keep])
            olab[sl] = jnp.where(m_isfg, labv, 0.0)
            ogx1[sl] = plsc.load_gather(lgx1, [amk])
            ogy1[sl] = plsc.load_gather(lgy1, [amk])
            ogx2[sl] = plsc.load_gather(lgx2, [amk])
            ogy2[sl] = plsc.load_gather(lgy2, [amk])

        ocps = [pltpu.async_copy(ox1, selo.at[0 * B + img], sem2),
                pltpu.async_copy(oy1, selo.at[1 * B + img], sem2),
                pltpu.async_copy(ox2, selo.at[2 * B + img], sem2),
                pltpu.async_copy(oy2, selo.at[3 * B + img], sem2),
                pltpu.async_copy(olab, selo.at[4 * B + img], sem2),
                pltpu.async_copy(ogx1, selo.at[5 * B + img], sem2),
                pltpu.async_copy(ogy1, selo.at[6 * B + img], sem2),
                pltpu.async_copy(ogx2, selo.at[7 * B + img], sem2),
                pltpu.async_copy(ogy2, selo.at[8 * B + img], sem2)]
        for cp in ocps:
            cp.wait()


def _make_sel_call():
    f32 = jnp.float32
    i32 = jnp.int32
    out = [jax.ShapeDtypeStruct((9 * B, ROIS_PER_IMAGE), f32)]
    scratch = [
        pltpu.VMEM((NPAD,), f32),
        pltpu.VMEM((NPAD,), i32),
        pltpu.VMEM((NPAD,), f32),
        pltpu.VMEM((NPAD,), f32),
        pltpu.VMEM((NPAD,), f32),
        pltpu.VMEM((NPAD,), f32),
        pltpu.VMEM((NPAD,), i32),
        pltpu.VMEM((KPAD,), f32),
        pltpu.VMEM((KPAD,), f32),
        pltpu.VMEM((KPAD,), f32),
        pltpu.VMEM((KPAD,), f32),
        pltpu.VMEM((KPAD,), f32),
        pltpu.VMEM((FG_ROIS,), i32),
        pltpu.VMEM((ROIS_PER_IMAGE,), i32),
        pltpu.VMEM((ROIS_PER_IMAGE,), f32),
        pltpu.VMEM((ROIS_PER_IMAGE,), f32),
        pltpu.VMEM((ROIS_PER_IMAGE,), f32),
        pltpu.VMEM((ROIS_PER_IMAGE,), f32),
        pltpu.VMEM((ROIS_PER_IMAGE,), f32),
        pltpu.VMEM((ROIS_PER_IMAGE,), f32),
        pltpu.VMEM((ROIS_PER_IMAGE,), f32),
        pltpu.VMEM((ROIS_PER_IMAGE,), f32),
        pltpu.VMEM((ROIS_PER_IMAGE,), f32),
        pltpu.SemaphoreType.DMA,
        pltpu.SemaphoreType.DMA,
    ]
    mesh = plsc.VectorSubcoreMesh(core_axis_name="c", subcore_axis_name="s",
                                  num_cores=1, num_subcores=16)
    return pl.kernel(_sel_body, out_type=out, mesh=mesh, scratch_types=scratch,
                     compiler_params=pltpu.CompilerParams(needs_layout_passes=False))


def _tc_body(selp, rois, labels, tgts, ins, outs):
    x1 = selp[0 * B:1 * B]
    y1 = selp[1 * B:2 * B]
    x2 = selp[2 * B:3 * B]
    y2 = selp[3 * B:4 * B]
    ew = x2 - x1 + 1.0
    eh = y2 - y1 + 1.0
    ecx = x1 + 0.5 * ew
    ecy = y1 + 0.5 * eh
    g1 = selp[5 * B:6 * B]
    h1 = selp[6 * B:7 * B]
    g2 = selp[7 * B:8 * B]
    h2 = selp[8 * B:9 * B]
    gw = g2 - g1 + 1.0
    gh = h2 - h1 + 1.0
    gcx = g1 + 0.5 * gw
    gcy = h1 + 0.5 * gh
    dx = ((gcx - ecx) / ew) / STDS[0]
    dy = ((gcy - ecy) / eh) / STDS[1]
    dw = jnp.log(gw / ew) / STDS[2]
    dh = jnp.log(gh / eh) / STDS[3]
    lb = selp[4 * B:5 * B]
    fg = lb > 0.0
    w = jnp.where(fg, 1.0, 0.0)
    col0 = lax.broadcasted_iota(jnp.int32, (B, ROIS_PER_IMAGE), 0).astype(jnp.float32)
    rois[...] = jnp.stack([col0, x1, y1, x2, y2], axis=-1)
    labels[...] = lb
    tgts[...] = jnp.stack([jnp.where(fg, dx, 0.0), jnp.where(fg, dy, 0.0),
                           jnp.where(fg, dw, 0.0), jnp.where(fg, dh, 0.0)],
                          axis=-1)
    wh = jnp.stack([w, w, w, w], axis=-1)
    ins[...] = wh
    outs[...] = wh


def kernel(all_rois, gt_boxes, num_boxes):
    f32 = jnp.float32
    pad = ((0, 0), (0, NPAD - NTOT))
    rx1 = jnp.pad(jnp.concatenate([all_rois[:, :, 1], gt_boxes[:, :, 0]], axis=1), pad)
    ry1 = jnp.pad(jnp.concatenate([all_rois[:, :, 2], gt_boxes[:, :, 1]], axis=1), pad)
    rx2 = jnp.pad(jnp.concatenate([all_rois[:, :, 3], gt_boxes[:, :, 2]], axis=1), pad)
    ry2 = jnp.pad(jnp.concatenate([all_rois[:, :, 4], gt_boxes[:, :, 3]], axis=1), pad)
    kp = ((0, 0), (1, KPAD - K - 1))
    gx1 = jnp.pad(gt_boxes[:, :, 0], kp)
    gy1 = jnp.pad(gt_boxes[:, :, 1], kp)
    gx2 = jnp.pad(gt_boxes[:, :, 2], kp)
    gy2 = jnp.pad(gt_boxes[:, :, 3], kp)
    glab = jnp.pad(gt_boxes[:, :, 4], kp)
    perm = jnp.asarray(_PERMS)

    rq = [a.reshape(NROW, 128) for a in (rx1, ry1, rx2, ry2)]
    ge = [jnp.repeat(a, LROW, axis=0) for a in (gx1, gy1, gx2, gy2)]
    pmq, amq = pl.pallas_call(
        _iou_body,
        out_shape=[jax.ShapeDtypeStruct((NROW, 128), f32),
                   jax.ShapeDtypeStruct((NROW, 128), jnp.int32)])(*rq, *ge)
    pm = pmq.reshape(B, NPAD)
    am = amq.reshape(B, NPAD)

    sel = _make_sel_call()
    (selp,) = sel(pm, am, rx1, ry1, rx2, ry2, perm, gx1, gy1, gx2, gy2, glab)

    rois, labels, bbox_targets, bbox_inside, bbox_outside = pl.pallas_call(
        _tc_body,
        out_shape=[jax.ShapeDtypeStruct((B, ROIS_PER_IMAGE, 5), f32),
                   jax.ShapeDtypeStruct((B, ROIS_PER_IMAGE), f32),
                   jax.ShapeDtypeStruct((B, ROIS_PER_IMAGE, 4), f32),
                   jax.ShapeDtypeStruct((B, ROIS_PER_IMAGE, 4), f32),
                   jax.ShapeDtypeStruct((B, ROIS_PER_IMAGE, 4), f32)],
    )(selp)
    return rois, labels, bbox_targets, bbox_inside, bbox_outside
